# baseline (speedup 1.0000x reference)
import jax
import jax.numpy as jnp
from jax.experimental import pallas as pl
from jax.experimental.pallas import tpu as pltpu

HASH_DIM = 8
N_HASHES = 4


def _prep_kernel(w_ref, rot_ref, u_ref, c_ref):
    w = w_ref[...]
    uvec = u_ref[0, :]
    uh = uvec / (jnp.sqrt(jnp.sum(uvec * uvec)) + 1e-6)
    hm = jnp.eye(HASH_DIM, dtype=jnp.float32) - 2.0 * uh[:, None] * uh[None, :]
    for h in range(N_HASHES):
        mh = jax.lax.dot_general(
            w, rot_ref[h],
            dimension_numbers=(((0,), (0,)), ((), ())),
            preferred_element_type=jnp.float32,
        )
        c_ref[:, h * HASH_DIM:(h + 1) * HASH_DIM] = mh
        c_ref[:, 32 + h * HASH_DIM:32 + (h + 1) * HASH_DIM] = jnp.dot(
            mh, hm, preferred_element_type=jnp.float32)


def _mm_kernel(x_ref, c_ref, out_ref):
    out_ref[...] = jnp.dot(x_ref[...], c_ref[...],
                           preferred_element_type=jnp.float32)


def kernel(x, W_base, rot, u):
    B, N, D = x.shape
    T = B * N
    x2 = x.reshape(T, D)

    c = pl.pallas_call(
        _prep_kernel,
        out_shape=jax.ShapeDtypeStruct((D, 8 * HASH_DIM), jnp.float32),
    )(W_base, rot, u.reshape(1, HASH_DIM))

    blk = 2048
    out = pl.pallas_call(
        _mm_kernel,
        grid=(T // blk,),
        in_specs=[
            pl.BlockSpec((blk, D), lambda i: (i, 0)),
            pl.BlockSpec((D, 8 * HASH_DIM), lambda i: (0, 0)),
        ],
        out_specs=pl.BlockSpec((blk, 8 * HASH_DIM), lambda i: (i, 0)),
        out_shape=jax.ShapeDtypeStruct((T, 8 * HASH_DIM), jnp.float32),
        compiler_params=pltpu.CompilerParams(
            dimension_semantics=("parallel",)),
    )(x2, c)

    sim = out[:, :32].reshape(B, N, N_HASHES, HASH_DIM)
    ort = out[:, 32:].reshape(B, N, N_HASHES, HASH_DIM)
    return (sim, ort)


# speedup vs baseline: 1.1457x; 1.1457x over previous
import jax
import jax.numpy as jnp
from jax.experimental import pallas as pl
from jax.experimental.pallas import tpu as pltpu

HASH_DIM = 8
N_HASHES = 4


def _fused_kernel(x_ref, w_ref, rot_ref, u_ref, sim_ref, ort_ref):
    w = w_ref[...]
    uvec = u_ref[0, :]
    uh = uvec / (jnp.sqrt(jnp.sum(uvec * uvec)) + 1e-6)
    hm = jnp.eye(HASH_DIM, dtype=jnp.float32) - 2.0 * uh[:, None] * uh[None, :]
    rcat = jnp.concatenate([rot_ref[h] for h in range(N_HASHES)], axis=1)
    hcat = jnp.concatenate([rot_ref[h] @ hm for h in range(N_HASHES)], axis=1)
    dn = (((0,), (0,)), ((), ()))
    csim = jax.lax.dot_general(w, rcat, dimension_numbers=dn,
                               preferred_element_type=jnp.float32)
    cort = jax.lax.dot_general(w, hcat, dimension_numbers=dn,
                               preferred_element_type=jnp.float32)
    xb = x_ref[...]
    sim_ref[...] = jnp.dot(xb, csim, preferred_element_type=jnp.float32)
    ort_ref[...] = jnp.dot(xb, cort, preferred_element_type=jnp.float32)


def kernel(x, W_base, rot, u):
    B, N, D = x.shape
    T = B * N
    x2 = x.reshape(T, D)
    cols = N_HASHES * HASH_DIM

    blk = 4096
    sim2, ort2 = pl.pallas_call(
        _fused_kernel,
        grid=(T // blk,),
        in_specs=[
            pl.BlockSpec((blk, D), lambda i: (i, 0)),
            pl.BlockSpec((HASH_DIM, D), lambda i: (0, 0)),
            pl.BlockSpec((N_HASHES, HASH_DIM, HASH_DIM), lambda i: (0, 0, 0)),
            pl.BlockSpec((1, HASH_DIM), lambda i: (0, 0)),
        ],
        out_specs=[
            pl.BlockSpec((blk, cols), lambda i: (i, 0)),
            pl.BlockSpec((blk, cols), lambda i: (i, 0)),
        ],
        out_shape=[
            jax.ShapeDtypeStruct((T, cols), jnp.float32),
            jax.ShapeDtypeStruct((T, cols), jnp.float32),
        ],
        compiler_params=pltpu.CompilerParams(
            dimension_semantics=("parallel",)),
    )(x2, W_base, rot, u.reshape(1, HASH_DIM))

    sim = sim2.reshape(B, N, N_HASHES, HASH_DIM)
    ort = ort2.reshape(B, N, N_HASHES, HASH_DIM)
    return (sim, ort)


# speedup vs baseline: 1.2063x; 1.0529x over previous
import jax
import jax.numpy as jnp
from jax.experimental import pallas as pl
from jax.experimental.pallas import tpu as pltpu

HASH_DIM = 8
N_HASHES = 4


def _fused_kernel(x_ref, w_ref, rot_ref, u_ref, sim_ref, ort_ref):
    w = w_ref[...]
    uvec = u_ref[0, :]
    uh = uvec / (jnp.sqrt(jnp.sum(uvec * uvec)) + 1e-6)
    hm = jnp.eye(HASH_DIM, dtype=jnp.float32) - 2.0 * uh[:, None] * uh[None, :]
    rcat = jnp.concatenate([rot_ref[h] for h in range(N_HASHES)], axis=1)
    hcat = jnp.concatenate([rot_ref[h] @ hm for h in range(N_HASHES)], axis=1)
    dn = (((0,), (0,)), ((), ()))
    ccat = jax.lax.dot_general(w, jnp.concatenate([rcat, hcat], axis=1),
                               dimension_numbers=dn,
                               preferred_element_type=jnp.float32)
    out = jnp.dot(x_ref[...], ccat, preferred_element_type=jnp.float32)
    sim_ref[...] = out[:, :N_HASHES * HASH_DIM]
    ort_ref[...] = out[:, N_HASHES * HASH_DIM:]


def kernel(x, W_base, rot, u):
    B, N, D = x.shape
    T = B * N
    x2 = x.reshape(T, D)
    cols = N_HASHES * HASH_DIM

    blk = 4096
    sim2, ort2 = pl.pallas_call(
        _fused_kernel,
        grid=(T // blk,),
        in_specs=[
            pl.BlockSpec((blk, D), lambda i: (i, 0)),
            pl.BlockSpec((HASH_DIM, D), lambda i: (0, 0)),
            pl.BlockSpec((N_HASHES, HASH_DIM, HASH_DIM), lambda i: (0, 0, 0)),
            pl.BlockSpec((1, HASH_DIM), lambda i: (0, 0)),
        ],
        out_specs=[
            pl.BlockSpec((blk, cols), lambda i: (i, 0)),
            pl.BlockSpec((blk, cols), lambda i: (i, 0)),
        ],
        out_shape=[
            jax.ShapeDtypeStruct((T, cols), jnp.float32),
            jax.ShapeDtypeStruct((T, cols), jnp.float32),
        ],
        compiler_params=pltpu.CompilerParams(
            dimension_semantics=("parallel",)),
    )(x2, W_base, rot, u.reshape(1, HASH_DIM))

    sim = sim2.reshape(B, N, N_HASHES, HASH_DIM)
    ort = ort2.reshape(B, N, N_HASHES, HASH_DIM)
    return (sim, ort)


# speedup vs baseline: 1.2064x; 1.0001x over previous
import jax
import jax.numpy as jnp
from jax.experimental import pallas as pl
from jax.experimental.pallas import tpu as pltpu

HASH_DIM = 8
N_HASHES = 4
K_STREAMS = 8
BLK = 512


def _fused_kernel(*refs):
    x_refs = refs[:K_STREAMS]
    w_ref, rot_ref, u_ref, sim_ref, ort_ref = refs[K_STREAMS:]
    w = w_ref[...]
    uvec = u_ref[0, :]
    uh = uvec / (jnp.sqrt(jnp.sum(uvec * uvec)) + 1e-6)
    hm = jnp.eye(HASH_DIM, dtype=jnp.float32) - 2.0 * uh[:, None] * uh[None, :]
    cats = [rot_ref[h] for h in range(N_HASHES)]
    cats += [rot_ref[h] @ hm for h in range(N_HASHES)]
    ccat = jax.lax.dot_general(
        w, jnp.concatenate(cats, axis=1),
        dimension_numbers=(((0,), (0,)), ((), ())),
        preferred_element_type=jnp.float32)
    half = N_HASHES * HASH_DIM
    for j in range(K_STREAMS):
        out = jnp.dot(x_refs[j][...], ccat, preferred_element_type=jnp.float32)
        sim_ref[j * BLK:(j + 1) * BLK, :] = out[:, :half]
        ort_ref[j * BLK:(j + 1) * BLK, :] = out[:, half:]


def kernel(x, W_base, rot, u):
    B, N, D = x.shape
    T = B * N
    x2 = x.reshape(T, D)
    cols = N_HASHES * HASH_DIM
    step_rows = K_STREAMS * BLK

    def x_spec(j):
        return pl.BlockSpec((BLK, D), lambda i, j=j: (i * K_STREAMS + j, 0))

    sim2, ort2 = pl.pallas_call(
        _fused_kernel,
        grid=(T // step_rows,),
        in_specs=[x_spec(j) for j in range(K_STREAMS)] + [
            pl.BlockSpec((HASH_DIM, D), lambda i: (0, 0)),
            pl.BlockSpec((N_HASHES, HASH_DIM, HASH_DIM), lambda i: (0, 0, 0)),
            pl.BlockSpec((1, HASH_DIM), lambda i: (0, 0)),
        ],
        out_specs=[
            pl.BlockSpec((step_rows, cols), lambda i: (i, 0)),
            pl.BlockSpec((step_rows, cols), lambda i: (i, 0)),
        ],
        out_shape=[
            jax.ShapeDtypeStruct((T, cols), jnp.float32),
            jax.ShapeDtypeStruct((T, cols), jnp.float32),
        ],
        compiler_params=pltpu.CompilerParams(
            dimension_semantics=("arbitrary",)),
    )(*([x2] * K_STREAMS), W_base, rot, u.reshape(1, HASH_DIM))

    sim = sim2.reshape(B, N, N_HASHES, HASH_DIM)
    ort = ort2.reshape(B, N, N_HASHES, HASH_DIM)
    return (sim, ort)
